# Initial kernel scaffold; baseline (speedup 1.0000x reference)
import jax
import jax.numpy as jnp
from jax.experimental import pallas as pl


def kernel(embeddings, edge_index):
    raise NotImplementedError("write your pallas kernel here")


# speedup vs baseline: 3.8980x; 3.8980x over previous
import jax
import jax.numpy as jnp
from jax import lax
from jax.experimental import pallas as pl
from jax.experimental.pallas import tpu as pltpu
from jax.experimental.pallas import tpu_sc as plsc

N = 100000
D = 32
E = 1600000
LAYERS = 3

NC, NS, L = 2, 16, 16
NW = NC * NS
H = D // NC
NPAD = 100352
EPAD = NW * 50176
ER = EPAD // 128
TROWS = NPAD // NS
CH = 1024
NCHUNK = (EPAD // NS) // CH
WCHUNK = (EPAD // NW) // CH
DRS = 1568

f32 = jnp.float32
i32 = jnp.int32


def _c32(v):
    return jnp.asarray(v, dtype=i32)


def _axis(name):
    return _c32(lax.axis_index(name))


_mesh = plsc.VectorSubcoreMesh(core_axis_name="c", subcore_axis_name="s")
_sc_params = pltpu.CompilerParams(needs_layout_passes=False,
                                  use_tc_tiling_on_sc=False)


def _fill(ref, rows, value):
    v = jnp.full((L,), value, dtype=ref.dtype)

    @pl.loop(_c32(0), _c32(rows))
    def _(i):
        ref[pl.ds(_c32(i) * _c32(L), L)] = v


def _deg_body(dst_hbm, deg_out, acc, zbuf, idxv, onesv, sem):
    c = _axis("c")
    s = _axis("s")
    w = s * _c32(NC) + c

    _fill(zbuf, TROWS // L, 0.0)
    _fill(onesv, CH // L, 1.0)
    pltpu.sync_copy(zbuf, acc.at[pl.ds(s * _c32(TROWS), TROWS)])
    plsc.subcore_barrier()

    @pl.loop(_c32(0), _c32(WCHUNK))
    def _(m):
        e0 = w * _c32(WCHUNK * CH) + _c32(m) * _c32(CH)
        pltpu.sync_copy(dst_hbm.at[pl.ds(e0, CH)], idxv)
        pltpu.sync_copy(onesv, acc.at[idxv], add=True)

    plsc.subcore_barrier()
    pltpu.sync_copy(acc.at[pl.ds(s * _c32(TROWS), TROWS)], zbuf)
    pltpu.sync_copy(zbuf, deg_out.at[c, pl.ds(s * _c32(TROWS), TROWS)])


_k_deg = pl.kernel(
    _deg_body,
    out_type=jax.ShapeDtypeStruct((NC, NPAD), f32),
    mesh=_mesh,
    compiler_params=_sc_params,
    scratch_types=[
        pltpu.VMEM_SHARED((NPAD,), f32),
        pltpu.VMEM((TROWS,), f32),
        pltpu.VMEM((CH,), i32),
        pltpu.VMEM((CH,), f32),
        pltpu.SemaphoreType.DMA,
    ],
)


def _dinv_body(deg_ref, dinv_ref):
    deg = deg_ref[0] + deg_ref[1]
    dinv_ref[...] = jnp.where(deg > 0.0, lax.rsqrt(deg), 0.0)


_k_dinv = pl.pallas_call(
    _dinv_body,
    out_shape=jax.ShapeDtypeStruct((NPAD // 128, 128), f32),
)


def _norm_body(src_hbm, dst_hbm, dinv_hbm, norm_out, sidx, didx, sg, dg, sem):
    c = _axis("c")
    s = _axis("s")
    w = s * _c32(NC) + c

    @pl.loop(_c32(0), _c32(WCHUNK))
    def _(m):
        e0 = w * _c32(WCHUNK * CH) + _c32(m) * _c32(CH)
        pltpu.sync_copy(src_hbm.at[pl.ds(e0, CH)], sidx)
        pltpu.sync_copy(dst_hbm.at[pl.ds(e0, CH)], didx)
        d1 = pltpu.async_copy(dinv_hbm.at[sidx], sg, sem)
        d2 = pltpu.async_copy(dinv_hbm.at[didx], dg, sem)
        d1.wait()
        d2.wait()

        @pl.loop(_c32(0), _c32(CH // L))
        def _(v):
            o = _c32(v) * _c32(L)
            sg[pl.ds(o, L)] = sg[pl.ds(o, L)] * dg[pl.ds(o, L)]

        pltpu.sync_copy(sg, norm_out.at[pl.ds(e0, CH)])


_k_norm = pl.kernel(
    _norm_body,
    out_type=jax.ShapeDtypeStruct((EPAD,), f32),
    mesh=_mesh,
    compiler_params=_sc_params,
    scratch_types=[
        pltpu.VMEM((CH,), i32),
        pltpu.VMEM((CH,), i32),
        pltpu.VMEM((CH,), f32),
        pltpu.VMEM((CH,), f32),
        pltpu.SemaphoreType.DMA,
    ],
)


def _layer_body(src_hbm, dst_hbm, norm_hbm, x_all, s_prev, x_next, s_next,
                acc, sidx, didx, normv, gcols, abuf, sbuf, sem):
    c = _axis("c")
    s = _axis("s")

    _fill(abuf, DRS // L, 0.0)
    for m in range(NPAD // DRS):
        pltpu.sync_copy(abuf, acc.at[pl.ds(s * _c32(NPAD) + _c32(m * DRS), DRS)])
    plsc.subcore_barrier()

    @pl.loop(_c32(0), _c32(NCHUNK))
    def _(m):
        e0 = s * _c32(NCHUNK * CH) + _c32(m) * _c32(CH)
        pltpu.sync_copy(src_hbm.at[pl.ds(e0, CH)], sidx)
        pltpu.sync_copy(dst_hbm.at[pl.ds(e0, CH)], didx)
        pltpu.sync_copy(norm_hbm.at[pl.ds(e0, CH)], normv)
        gds = [
            pltpu.async_copy(
                x_all.at[pl.ds((c * _c32(H) + _c32(d)) * _c32(NPAD), NPAD)]
                .at[sidx],
                gcols.at[pl.ds(d * CH, CH)], sem)
            for d in range(H)
        ]
        for g in gds:
            g.wait()

        @pl.loop(_c32(0), _c32(CH // L))
        def _(v):
            o = _c32(v) * _c32(L)
            nv = normv[pl.ds(o, L)]
            for d in range(H):
                gcols[pl.ds(_c32(d * CH) + o, L)] = (
                    gcols[pl.ds(_c32(d * CH) + o, L)] * nv)

        sds = [
            pltpu.async_copy(
                gcols.at[pl.ds(d * CH, CH)],
                acc.at[pl.ds(_c32(d) * _c32(NPAD), NPAD)].at[didx],
                sem, add=True)
            for d in range(H)
        ]
        for g in sds:
            g.wait()

    plsc.subcore_barrier()

    for d in range(H):
        for m in range(TROWS // DRS):
            r0 = s * _c32(TROWS) + _c32(m * DRS)
            lo = _c32(d) * _c32(NPAD) + r0
            go = (c * _c32(H) + _c32(d)) * _c32(NPAD) + r0
            pltpu.sync_copy(acc.at[pl.ds(lo, DRS)], abuf)
            pltpu.sync_copy(s_prev.at[pl.ds(go, DRS)], sbuf)

            @pl.loop(_c32(0), _c32(DRS // L))
            def _(v):
                o = _c32(v) * _c32(L)
                sbuf[pl.ds(o, L)] = sbuf[pl.ds(o, L)] + abuf[pl.ds(o, L)]

            pltpu.sync_copy(abuf, x_next.at[pl.ds(go, DRS)])
            pltpu.sync_copy(sbuf, s_next.at[pl.ds(go, DRS)])


_k_layer = pl.kernel(
    _layer_body,
    out_type=(jax.ShapeDtypeStruct((D * NPAD,), f32),
              jax.ShapeDtypeStruct((D * NPAD,), f32)),
    mesh=_mesh,
    compiler_params=_sc_params,
    scratch_types=[
        pltpu.VMEM_SHARED((H * NPAD,), f32),
        pltpu.VMEM((CH,), i32),
        pltpu.VMEM((CH,), i32),
        pltpu.VMEM((CH,), f32),
        pltpu.VMEM((H * CH,), f32),
        pltpu.VMEM((DRS,), f32),
        pltpu.VMEM((DRS,), f32),
        pltpu.SemaphoreType.DMA,
    ],
)


def _dot_body(src_hbm, dst_hbm, s_all, dots, sidx, didx, sgall, dgall, obuf,
              sem):
    c = _axis("c")
    s = _axis("s")

    @pl.loop(_c32(0), _c32(NCHUNK))
    def _(m):
        e0 = s * _c32(NCHUNK * CH) + _c32(m) * _c32(CH)
        pltpu.sync_copy(src_hbm.at[pl.ds(e0, CH)], sidx)
        pltpu.sync_copy(dst_hbm.at[pl.ds(e0, CH)], didx)
        ds_ = [
            pltpu.async_copy(
                s_all.at[pl.ds((c * _c32(H) + _c32(d)) * _c32(NPAD), NPAD)]
                .at[sidx],
                sgall.at[pl.ds(d * CH, CH)], sem)
            for d in range(H)
        ] + [
            pltpu.async_copy(
                s_all.at[pl.ds((c * _c32(H) + _c32(d)) * _c32(NPAD), NPAD)]
                .at[didx],
                dgall.at[pl.ds(d * CH, CH)], sem)
            for d in range(H)
        ]
        for g in ds_:
            g.wait()

        @pl.loop(_c32(0), _c32(CH // L))
        def _(v):
            o = _c32(v) * _c32(L)
            av = sgall[pl.ds(o, L)] * dgall[pl.ds(o, L)]
            for d in range(1, H):
                av = av + (sgall[pl.ds(_c32(d * CH) + o, L)]
                           * dgall[pl.ds(_c32(d * CH) + o, L)])
            obuf[pl.ds(o, L)] = av

        pltpu.sync_copy(obuf, dots.at[pl.ds(c * _c32(EPAD) + e0, CH)])


_k_dot = pl.kernel(
    _dot_body,
    out_type=jax.ShapeDtypeStruct((NC * EPAD,), f32),
    mesh=_mesh,
    compiler_params=_sc_params,
    scratch_types=[
        pltpu.VMEM((CH,), i32),
        pltpu.VMEM((CH,), i32),
        pltpu.VMEM((H * CH,), f32),
        pltpu.VMEM((H * CH,), f32),
        pltpu.VMEM((CH,), f32),
        pltpu.SemaphoreType.DMA,
    ],
)


def _combine_body(parts_ref, out_ref):
    out_ref[...] = 0.0625 * (parts_ref[0] + parts_ref[1])


_k_combine = pl.pallas_call(
    _combine_body,
    out_shape=jax.ShapeDtypeStruct((ER, 128), f32),
)


def kernel(embeddings, edge_index):
    src = edge_index[0].astype(i32)
    dst = edge_index[1].astype(i32)
    srcp = jnp.concatenate([src, jnp.zeros((EPAD - E,), i32)])
    dstp = jnp.concatenate([dst, jnp.full((EPAD - E,), N, i32)])

    deg = _k_deg(dstp)
    dinv = _k_dinv(deg.reshape(NC, NPAD // 128, 128)).reshape(NPAD)
    norm = _k_norm(srcp, dstp, dinv)

    embp = jnp.pad(embeddings.astype(f32), ((0, NPAD - N), (0, 0)))
    x_all = embp.T.reshape(D * NPAD)
    s_all = x_all
    for _ in range(LAYERS):
        x_all, s_all = _k_layer(srcp, dstp, norm, x_all, s_all)

    dots = _k_dot(srcp, dstp, s_all)
    out = _k_combine(dots.reshape(NC, ER, 128))
    return out.reshape(EPAD)[:E]


# speedup vs baseline: 4.2312x; 1.0855x over previous
import jax
import jax.numpy as jnp
from jax import lax
from jax.experimental import pallas as pl
from jax.experimental.pallas import tpu as pltpu
from jax.experimental.pallas import tpu_sc as plsc

N = 100000
D = 32
E = 1600000
LAYERS = 3

NC, NS, L = 2, 16, 16
NW = NC * NS
H = D // NC
NPAD = 100352
EPAD = NW * 50176
ER = EPAD // 128
TROWS = NPAD // NS
CH = 512
NCHUNK = (EPAD // NS) // CH
WCHUNK = (EPAD // NW) // CH
DRS = 1568

f32 = jnp.float32
i32 = jnp.int32


def _c32(v):
    return jnp.asarray(v, dtype=i32)


def _axis(name):
    return _c32(lax.axis_index(name))


_mesh = plsc.VectorSubcoreMesh(core_axis_name="c", subcore_axis_name="s")
_sc_params = pltpu.CompilerParams(needs_layout_passes=False,
                                  use_tc_tiling_on_sc=False)


def _fill(ref, rows, value):
    v = jnp.full((L,), value, dtype=ref.dtype)

    @pl.loop(_c32(0), _c32(rows))
    def _(i):
        ref[pl.ds(_c32(i) * _c32(L), L)] = v


def _deg_body(dst_hbm, deg_out, acc, zbuf, idxv, onesv, sem):
    c = _axis("c")
    s = _axis("s")
    w = s * _c32(NC) + c

    _fill(zbuf, TROWS // L, 0.0)
    _fill(onesv, CH // L, 1.0)
    pltpu.sync_copy(zbuf, acc.at[pl.ds(s * _c32(TROWS), TROWS)])
    plsc.subcore_barrier()

    @pl.loop(_c32(0), _c32(WCHUNK))
    def _(m):
        e0 = w * _c32(WCHUNK * CH) + _c32(m) * _c32(CH)
        pltpu.sync_copy(dst_hbm.at[pl.ds(e0, CH)], idxv)
        pltpu.sync_copy(onesv, acc.at[idxv], add=True)

    plsc.subcore_barrier()
    pltpu.sync_copy(acc.at[pl.ds(s * _c32(TROWS), TROWS)], zbuf)
    pltpu.sync_copy(zbuf, deg_out.at[c, pl.ds(s * _c32(TROWS), TROWS)])


_k_deg = pl.kernel(
    _deg_body,
    out_type=jax.ShapeDtypeStruct((NC, NPAD), f32),
    mesh=_mesh,
    compiler_params=_sc_params,
    scratch_types=[
        pltpu.VMEM_SHARED((NPAD,), f32),
        pltpu.VMEM((TROWS,), f32),
        pltpu.VMEM((CH,), i32),
        pltpu.VMEM((CH,), f32),
        pltpu.SemaphoreType.DMA,
    ],
)


def _dinv_body(deg_ref, dinv_ref):
    deg = deg_ref[0] + deg_ref[1]
    dinv_ref[...] = jnp.where(deg > 0.0, lax.rsqrt(deg), 0.0)


_k_dinv = pl.pallas_call(
    _dinv_body,
    out_shape=jax.ShapeDtypeStruct((NPAD // 128, 128), f32),
)


def _norm_body(src_hbm, dst_hbm, dinv_hbm, norm_out, sidx, didx, sg, dg, sem):
    c = _axis("c")
    s = _axis("s")
    w = s * _c32(NC) + c

    @pl.loop(_c32(0), _c32(WCHUNK))
    def _(m):
        e0 = w * _c32(WCHUNK * CH) + _c32(m) * _c32(CH)
        pltpu.sync_copy(src_hbm.at[pl.ds(e0, CH)], sidx)
        pltpu.sync_copy(dst_hbm.at[pl.ds(e0, CH)], didx)
        d1 = pltpu.async_copy(dinv_hbm.at[sidx], sg, sem)
        d2 = pltpu.async_copy(dinv_hbm.at[didx], dg, sem)
        d1.wait()
        d2.wait()

        @pl.loop(_c32(0), _c32(CH // L))
        def _(v):
            o = _c32(v) * _c32(L)
            sg[pl.ds(o, L)] = sg[pl.ds(o, L)] * dg[pl.ds(o, L)]

        pltpu.sync_copy(sg, norm_out.at[pl.ds(e0, CH)])


_k_norm = pl.kernel(
    _norm_body,
    out_type=jax.ShapeDtypeStruct((EPAD,), f32),
    mesh=_mesh,
    compiler_params=_sc_params,
    scratch_types=[
        pltpu.VMEM((CH,), i32),
        pltpu.VMEM((CH,), i32),
        pltpu.VMEM((CH,), f32),
        pltpu.VMEM((CH,), f32),
        pltpu.SemaphoreType.DMA,
    ],
)


def _layer_body(src_hbm, dst_hbm, norm_hbm, x_all, s_prev, x_next, s_next,
                acc, sidxA, didxA, normA, gcA, sidxB, didxB, normB, gcB,
                abuf, sbuf, semg, sems):
    c = _axis("c")
    s = _axis("s")
    base = s * _c32(NCHUNK * CH)

    def stage(mi, sidx, didx, normv):
        e0 = base + mi * _c32(CH)
        pltpu.sync_copy(src_hbm.at[pl.ds(e0, CH)], sidx)
        pltpu.sync_copy(dst_hbm.at[pl.ds(e0, CH)], didx)
        pltpu.sync_copy(norm_hbm.at[pl.ds(e0, CH)], normv)

    def g_descs(sidx, gc, make):
        f = pltpu.make_async_copy if make else pltpu.async_copy
        return [
            f(x_all.at[pl.ds((c * _c32(H) + _c32(d)) * _c32(NPAD), NPAD)]
              .at[sidx],
              gc.at[pl.ds(d * CH, CH)], semg)
            for d in range(H)
        ]

    def s_descs(didx, gc, make):
        if make:
            return [
                pltpu.make_async_copy(
                    gc.at[pl.ds(d * CH, CH)],
                    acc.at[pl.ds(_c32(d) * _c32(NPAD), NPAD)].at[didx],
                    sems)
                for d in range(H)
            ]
        return [
            pltpu.async_copy(
                gc.at[pl.ds(d * CH, CH)],
                acc.at[pl.ds(_c32(d) * _c32(NPAD), NPAD)].at[didx],
                sems, add=True)
            for d in range(H)
        ]

    def wait(descs):
        for g in descs:
            g.wait()

    def mul(normv, gc):
        @pl.loop(_c32(0), _c32(CH // L))
        def _(v):
            o = _c32(v) * _c32(L)
            nv = normv[pl.ds(o, L)]
            for d in range(H):
                gc[pl.ds(_c32(d * CH) + o, L)] = (
                    gc[pl.ds(_c32(d * CH) + o, L)] * nv)

    _fill(abuf, DRS // L, 0.0)
    for m in range(NPAD // DRS):
        pltpu.sync_copy(abuf, acc.at[pl.ds(s * _c32(NPAD) + _c32(m * DRS), DRS)])
    plsc.subcore_barrier()

    stage(_c32(0), sidxA, didxA, normA)
    g_descs(sidxA, gcA, False)
    stage(_c32(1), sidxB, didxB, normB)
    g_descs(sidxB, gcB, False)

    @pl.loop(_c32(0), _c32(NCHUNK // 2 - 1))
    def _(h):
        m = _c32(h) * _c32(2)
        wait(g_descs(sidxA, gcA, True))
        mul(normA, gcA)
        s_descs(didxA, gcA, False)
        wait(g_descs(sidxB, gcB, True))
        mul(normB, gcB)
        s_descs(didxB, gcB, False)
        wait(s_descs(didxA, gcA, True))
        stage(m + _c32(2), sidxA, didxA, normA)
        g_descs(sidxA, gcA, False)
        wait(s_descs(didxB, gcB, True))
        stage(m + _c32(3), sidxB, didxB, normB)
        g_descs(sidxB, gcB, False)

    wait(g_descs(sidxA, gcA, True))
    mul(normA, gcA)
    s_descs(didxA, gcA, False)
    wait(g_descs(sidxB, gcB, True))
    mul(normB, gcB)
    s_descs(didxB, gcB, False)
    wait(s_descs(didxA, gcA, True))
    wait(s_descs(didxB, gcB, True))

    plsc.subcore_barrier()

    for d in range(H):
        for m in range(TROWS // DRS):
            r0 = s * _c32(TROWS) + _c32(m * DRS)
            lo = _c32(d) * _c32(NPAD) + r0
            go = (c * _c32(H) + _c32(d)) * _c32(NPAD) + r0
            pltpu.sync_copy(acc.at[pl.ds(lo, DRS)], abuf)
            pltpu.sync_copy(s_prev.at[pl.ds(go, DRS)], sbuf)

            @pl.loop(_c32(0), _c32(DRS // L))
            def _(v):
                o = _c32(v) * _c32(L)
                sbuf[pl.ds(o, L)] = sbuf[pl.ds(o, L)] + abuf[pl.ds(o, L)]

            pltpu.sync_copy(abuf, x_next.at[pl.ds(go, DRS)])
            pltpu.sync_copy(sbuf, s_next.at[pl.ds(go, DRS)])


_k_layer = pl.kernel(
    _layer_body,
    out_type=(jax.ShapeDtypeStruct((D * NPAD,), f32),
              jax.ShapeDtypeStruct((D * NPAD,), f32)),
    mesh=_mesh,
    compiler_params=_sc_params,
    scratch_types=[
        pltpu.VMEM_SHARED((H * NPAD,), f32),
        pltpu.VMEM((CH,), i32),
        pltpu.VMEM((CH,), i32),
        pltpu.VMEM((CH,), f32),
        pltpu.VMEM((H * CH,), f32),
        pltpu.VMEM((CH,), i32),
        pltpu.VMEM((CH,), i32),
        pltpu.VMEM((CH,), f32),
        pltpu.VMEM((H * CH,), f32),
        pltpu.VMEM((DRS,), f32),
        pltpu.VMEM((DRS,), f32),
        pltpu.SemaphoreType.DMA,
        pltpu.SemaphoreType.DMA,
    ],
)


def _dot_body(src_hbm, dst_hbm, s_all, dots, sidx, didx, sgall, dgall, obuf,
              sem):
    c = _axis("c")
    s = _axis("s")

    @pl.loop(_c32(0), _c32(NCHUNK))
    def _(m):
        e0 = s * _c32(NCHUNK * CH) + _c32(m) * _c32(CH)
        pltpu.sync_copy(src_hbm.at[pl.ds(e0, CH)], sidx)
        pltpu.sync_copy(dst_hbm.at[pl.ds(e0, CH)], didx)
        ds_ = [
            pltpu.async_copy(
                s_all.at[pl.ds((c * _c32(H) + _c32(d)) * _c32(NPAD), NPAD)]
                .at[sidx],
                sgall.at[pl.ds(d * CH, CH)], sem)
            for d in range(H)
        ] + [
            pltpu.async_copy(
                s_all.at[pl.ds((c * _c32(H) + _c32(d)) * _c32(NPAD), NPAD)]
                .at[didx],
                dgall.at[pl.ds(d * CH, CH)], sem)
            for d in range(H)
        ]
        for g in ds_:
            g.wait()

        @pl.loop(_c32(0), _c32(CH // L))
        def _(v):
            o = _c32(v) * _c32(L)
            av = sgall[pl.ds(o, L)] * dgall[pl.ds(o, L)]
            for d in range(1, H):
                av = av + (sgall[pl.ds(_c32(d * CH) + o, L)]
                           * dgall[pl.ds(_c32(d * CH) + o, L)])
            obuf[pl.ds(o, L)] = av

        pltpu.sync_copy(obuf, dots.at[pl.ds(c * _c32(EPAD) + e0, CH)])


_k_dot = pl.kernel(
    _dot_body,
    out_type=jax.ShapeDtypeStruct((NC * EPAD,), f32),
    mesh=_mesh,
    compiler_params=_sc_params,
    scratch_types=[
        pltpu.VMEM((CH,), i32),
        pltpu.VMEM((CH,), i32),
        pltpu.VMEM((H * CH,), f32),
        pltpu.VMEM((H * CH,), f32),
        pltpu.VMEM((CH,), f32),
        pltpu.SemaphoreType.DMA,
    ],
)


def _combine_body(parts_ref, out_ref):
    out_ref[...] = 0.0625 * (parts_ref[0] + parts_ref[1])


_k_combine = pl.pallas_call(
    _combine_body,
    out_shape=jax.ShapeDtypeStruct((ER, 128), f32),
)


def kernel(embeddings, edge_index):
    src = edge_index[0].astype(i32)
    dst = edge_index[1].astype(i32)
    srcp = jnp.concatenate([src, jnp.zeros((EPAD - E,), i32)])
    dstp = jnp.concatenate([dst, jnp.full((EPAD - E,), N, i32)])

    deg = _k_deg(dstp)
    dinv = _k_dinv(deg.reshape(NC, NPAD // 128, 128)).reshape(NPAD)
    norm = _k_norm(srcp, dstp, dinv)

    embp = jnp.pad(embeddings.astype(f32), ((0, NPAD - N), (0, 0)))
    x_all = embp.T.reshape(D * NPAD)
    s_all = x_all
    for _ in range(LAYERS):
        x_all, s_all = _k_layer(srcp, dstp, norm, x_all, s_all)

    dots = _k_dot(srcp, dstp, s_all)
    out = _k_combine(dots.reshape(NC, ER, 128))
    return out.reshape(EPAD)[:E]
